# baseline (speedup 1.0000x reference)
import functools

import jax
import jax.numpy as jnp
from jax import lax
from jax.experimental import pallas as pl
from jax.experimental.pallas import tpu as pltpu
from jax.experimental.pallas import tpu_sc as plsc

N_NODES = 10000
N_EDGES = 320000
D = 128
H = 4 * D

NC = 2
NS = 16
NW = NC * NS
EPW = N_EDGES // NW
K = 80
NBLK = EPW // K
GB = 8
NG = -(-NBLK // GB)
NBLK_PAD = NG * GB
N_PAD = 10240
ROWS_PER_TILE = N_PAD // NS
ZCH = ROWS_PER_TILE // K

_MESH = plsc.VectorSubcoreMesh(core_axis_name="c", subcore_axis_name="s")


@functools.partial(
    pl.kernel,
    out_type=jax.ShapeDtypeStruct((NC, N_PAD, D), jnp.float32),
    mesh=_MESH,
    scratch_types=[
        pltpu.VMEM((2, GB, K), jnp.int32),
        pltpu.VMEM((2, GB, K), jnp.int32),
        pltpu.VMEM((K, D), jnp.float32),
        pltpu.VMEM((K, D), jnp.float32),
        pltpu.VMEM((K, D), jnp.float32),
        pltpu.VMEM((K, D), jnp.float32),
        pltpu.VMEM_SHARED((N_PAD, D), jnp.float32),
        pltpu.SemaphoreType.DMA,
        pltpu.SemaphoreType.DMA,
        pltpu.SemaphoreType.DMA,
        pltpu.SemaphoreType.DMA,
        pltpu.SemaphoreType.DMA,
        pltpu.SemaphoreType.DMA,
        pltpu.SemaphoreType.DMA,
        pltpu.SemaphoreType.DMA,
        pltpu.SemaphoreType.DMA,
        pltpu.SemaphoreType.DMA,
        pltpu.SemaphoreType.DMA,
        pltpu.SemaphoreType.DMA,
        pltpu.SemaphoreType.DMA,
    ],
)
def _sc_aggregate(x_hbm, src_hbm, dst_hbm, ea_hbm, z_hbm, out_hbm,
                  sidx3, didx3, eb0, eb1, eb2, eb3, accum,
                  gs0, gs1, gs2, gs3, es0, es1, es2, es3,
                  ts0, ts1, ts2, ts3, isem):
    c = lax.axis_index("c")
    s = lax.axis_index("s")
    w = c * NS + s
    eb = (eb0, eb1, eb2, eb3)
    gs = (gs0, gs1, gs2, gs3)
    es = (es0, es1, es2, es3)
    ts = (ts0, ts1, ts2, ts3)

    pltpu.sync_copy(z_hbm, eb0)
    for kk in range(ZCH):
        pltpu.sync_copy(eb0, accum.at[pl.ds(s * ROWS_PER_TILE + kk * K, K)])
    plsc.subcore_barrier()

    class _Op:
        def __init__(self, src, dst, sem, add=False):
            self.src, self.dst, self.sem, self.add = src, dst, sem, add

        def start(self):
            pltpu.async_copy(self.src, self.dst, self.sem, add=self.add)

        def wait(self):
            pltpu.make_async_copy(self.src, self.dst, self.sem).wait()

    def gadd_desc(p, gp, j):
        return _Op(x_hbm.at[sidx3.at[gp, j]], eb[p], gs[p], add=True)

    def efetch_desc(jb, p):
        return _Op(ea_hbm.at[pl.ds(w * EPW + jb * K, K)], eb[p], es[p])

    def escat_desc(p, gp, j):
        return _Op(eb[p], accum.at[didx3.at[gp, j]], ts[p], add=True)

    pltpu.sync_copy(src_hbm.at[w, pl.ds(0, GB)], sidx3.at[0])
    pltpu.sync_copy(dst_hbm.at[w, pl.ds(0, GB)], didx3.at[0])

    def group(g, carry):
        gp = g % 2

        @pl.when(g > 0)
        def _():
            pltpu.make_async_copy(src_hbm.at[w, pl.ds(0, GB)],
                                  sidx3.at[gp], isem).wait()
            pltpu.make_async_copy(dst_hbm.at[w, pl.ds(0, GB)],
                                  didx3.at[gp], isem).wait()

        for j in range(GB):
            jb = g * GB + j
            p = j % 4
            p1 = (j - 1) % 4
            p2 = (j - 2) % 4
            prev_gp = gp if j > 0 else 1 - gp
            prev_j = (j - 1) % GB
            pp_gp = gp if j > 1 else 1 - gp
            pp_j = (j - 2) % GB

            @pl.when(jb < NBLK)
            def _():
                @pl.when(jb >= 4)
                def _():
                    escat_desc(p, gp, j).wait()

                efetch_desc(jb, p).start()

                @pl.when(jb >= 2)
                def _():
                    gadd_desc(p2, pp_gp, pp_j).wait()
                    escat_desc(p2, pp_gp, pp_j).start()

                @pl.when(jb >= 1)
                def _():
                    efetch_desc(jb - 1, p1).wait()
                    gadd_desc(p1, prev_gp, prev_j).start()

            if j == 4:
                @pl.when(g + 1 < NG)
                def _():
                    pltpu.async_copy(src_hbm.at[w, pl.ds((g + 1) * GB, GB)],
                                     sidx3.at[1 - gp], isem)
                    pltpu.async_copy(dst_hbm.at[w, pl.ds((g + 1) * GB, GB)],
                                     didx3.at[1 - gp], isem)
        return carry

    lax.fori_loop(0, NG, group, 0)

    efetch_desc(NBLK - 1, 0).wait()
    gadd_desc(0, 1, 4).start()
    gadd_desc(3, 1, 3).wait()
    escat_desc(3, 1, 3).start()
    gadd_desc(0, 1, 4).wait()
    escat_desc(0, 1, 4).start()
    escat_desc(1, 1, 1).wait()
    escat_desc(2, 1, 2).wait()
    escat_desc(3, 1, 3).wait()
    escat_desc(0, 1, 4).wait()
    plsc.subcore_barrier()

    pltpu.sync_copy(accum.at[pl.ds(s * ROWS_PER_TILE, ROWS_PER_TILE)],
                    out_hbm.at[c, pl.ds(s * ROWS_PER_TILE, ROWS_PER_TILE)])


_NB = 400
_GRID = N_NODES // _NB


def _mlp_body(u0_ref, u1_ref, x_ref, w1_ref, b1_ref, w2_ref, b2_ref,
              g_ref, bt_ref, o_ref):
    u = u0_ref[...] + u1_ref[...]
    h = jnp.dot(u, w1_ref[...], preferred_element_type=jnp.float32) + b1_ref[...]
    h = jnp.maximum(h, 0.0)
    h = jnp.dot(h, w2_ref[...], preferred_element_type=jnp.float32) + b2_ref[...]
    h = h + x_ref[...]
    mean = jnp.mean(h, axis=-1, keepdims=True)
    hc = h - mean
    var = jnp.mean(hc * hc, axis=-1, keepdims=True)
    o_ref[...] = hc * lax.rsqrt(var + 1e-5) * g_ref[...] + bt_ref[...]


_mlp = pl.pallas_call(
    _mlp_body,
    grid=(_GRID,),
    in_specs=[
        pl.BlockSpec((_NB, D), lambda i: (i, 0)),
        pl.BlockSpec((_NB, D), lambda i: (i, 0)),
        pl.BlockSpec((_NB, D), lambda i: (i, 0)),
        pl.BlockSpec((D, H), lambda i: (0, 0)),
        pl.BlockSpec((1, H), lambda i: (0, 0)),
        pl.BlockSpec((H, D), lambda i: (0, 0)),
        pl.BlockSpec((1, D), lambda i: (0, 0)),
        pl.BlockSpec((1, D), lambda i: (0, 0)),
        pl.BlockSpec((1, D), lambda i: (0, 0)),
    ],
    out_specs=pl.BlockSpec((_NB, D), lambda i: (i, 0)),
    out_shape=jax.ShapeDtypeStruct((N_NODES, D), jnp.float32),
)


def kernel(x, edge_index, edge_attr, W1, b1, W2, b2, gamma, beta):
    src = edge_index[0].astype(jnp.int32).reshape(NW, NBLK, K)
    dst = edge_index[1].astype(jnp.int32).reshape(NW, NBLK, K)
    pad = ((0, 0), (0, NBLK_PAD - NBLK), (0, 0))
    src = jnp.pad(src, pad)
    dst = jnp.pad(dst, pad)
    zeros = jnp.zeros((K, D), jnp.float32)
    u_parts = _sc_aggregate(x, src, dst, edge_attr, zeros)
    return _mlp(u_parts[0], u_parts[1], x,
                W1, b1.reshape(1, H), W2, b2.reshape(1, D),
                gamma.reshape(1, D), beta.reshape(1, D))


# speedup vs baseline: 1.1339x; 1.1339x over previous
import functools

import jax
import jax.numpy as jnp
from jax import lax
from jax.experimental import pallas as pl
from jax.experimental.pallas import tpu as pltpu
from jax.experimental.pallas import tpu_sc as plsc

N_NODES = 10000
N_EDGES = 320000
D = 128
H = 4 * D

NC = 2
NS = 16
NW = NC * NS
EPW = N_EDGES // NW
K = 80
NBLK = EPW // K
GB = 8
NG = -(-NBLK // GB)
NBLK_PAD = NG * GB
N_PAD = 10240
ROWS_PER_TILE = N_PAD // NS
ZCH = ROWS_PER_TILE // K

_MESH = plsc.VectorSubcoreMesh(core_axis_name="c", subcore_axis_name="s")


@functools.partial(
    pl.kernel,
    out_type=jax.ShapeDtypeStruct((NC, N_PAD, D), jnp.float32),
    mesh=_MESH,
    scratch_types=[
        pltpu.VMEM((2, GB, K), jnp.int32),
        pltpu.VMEM((2, GB, K), jnp.int32),
        pltpu.VMEM((K, D), jnp.float32),
        pltpu.VMEM((K, D), jnp.float32),
        pltpu.VMEM((K, D), jnp.float32),
        pltpu.VMEM((K, D), jnp.float32),
        pltpu.VMEM_SHARED((N_PAD, D), jnp.float32),
        pltpu.SemaphoreType.DMA,
        pltpu.SemaphoreType.DMA,
        pltpu.SemaphoreType.DMA,
        pltpu.SemaphoreType.DMA,
        pltpu.SemaphoreType.DMA,
        pltpu.SemaphoreType.DMA,
        pltpu.SemaphoreType.DMA,
        pltpu.SemaphoreType.DMA,
        pltpu.SemaphoreType.DMA,
        pltpu.SemaphoreType.DMA,
        pltpu.SemaphoreType.DMA,
        pltpu.SemaphoreType.DMA,
        pltpu.SemaphoreType.DMA,
    ],
)
def _sc_aggregate(x_hbm, src_hbm, dst_hbm, ea_hbm, z_hbm, out_hbm,
                  sidx3, didx3, eb0, eb1, eb2, eb3, accum,
                  gs0, gs1, gs2, gs3, es0, es1, es2, es3,
                  ts0, ts1, ts2, ts3, isem):
    c = lax.axis_index("c")
    s = lax.axis_index("s")
    w = c * NS + s
    eb = (eb0, eb1, eb2, eb3)
    gs = (gs0, gs1, gs2, gs3)
    es = (es0, es1, es2, es3)
    ts = (ts0, ts1, ts2, ts3)

    pltpu.sync_copy(z_hbm, eb0)
    for kk in range(ZCH):
        pltpu.sync_copy(eb0, accum.at[pl.ds(s * ROWS_PER_TILE + kk * K, K)])
    plsc.subcore_barrier()

    class _Op:
        def __init__(self, src, dst, sem, add=False):
            self.src, self.dst, self.sem, self.add = src, dst, sem, add

        def start(self):
            pltpu.async_copy(self.src, self.dst, self.sem, add=self.add)

        def wait(self):
            pltpu.make_async_copy(self.src, self.dst, self.sem).wait()

    def gadd_desc(p, gp, j):
        return _Op(x_hbm.at[sidx3.at[gp, j]], eb[p], gs[p], add=True)

    def efetch_desc(jb, p):
        return _Op(ea_hbm.at[pl.ds(w * EPW + jb * K, K)], eb[p], es[p])

    def escat_desc(p, gp, j):
        return _Op(eb[p], accum.at[didx3.at[gp, j]], ts[p], add=True)

    pltpu.sync_copy(src_hbm.at[w, pl.ds(0, GB)], sidx3.at[0])
    pltpu.sync_copy(dst_hbm.at[w, pl.ds(0, GB)], didx3.at[0])

    def group(g, carry):
        gp = g % 2

        @pl.when(g > 0)
        def _():
            pltpu.make_async_copy(src_hbm.at[w, pl.ds(0, GB)],
                                  sidx3.at[gp], isem).wait()
            pltpu.make_async_copy(dst_hbm.at[w, pl.ds(0, GB)],
                                  didx3.at[gp], isem).wait()

        for j in range(GB):
            jb = g * GB + j
            p = j % 4
            p1 = (j - 1) % 4
            p2 = (j - 2) % 4
            prev_gp = gp if j > 0 else 1 - gp
            prev_j = (j - 1) % GB
            pp_gp = gp if j > 1 else 1 - gp
            pp_j = (j - 2) % GB

            @pl.when(jb < NBLK)
            def _():
                @pl.when(jb >= 4)
                def _():
                    escat_desc(p, gp, j).wait()

                efetch_desc(jb, p).start()

                @pl.when(jb >= 1)
                def _():
                    efetch_desc(jb - 1, p1).wait()
                    gadd_desc(p1, prev_gp, prev_j).start()

                @pl.when(jb >= 2)
                def _():
                    gadd_desc(p2, pp_gp, pp_j).wait()
                    escat_desc(p2, pp_gp, pp_j).start()

            if j == 4:
                @pl.when(g + 1 < NG)
                def _():
                    pltpu.async_copy(src_hbm.at[w, pl.ds((g + 1) * GB, GB)],
                                     sidx3.at[1 - gp], isem)
                    pltpu.async_copy(dst_hbm.at[w, pl.ds((g + 1) * GB, GB)],
                                     didx3.at[1 - gp], isem)
        return carry

    lax.fori_loop(0, NG, group, 0)

    efetch_desc(NBLK - 1, 0).wait()
    gadd_desc(0, 1, 4).start()
    gadd_desc(3, 1, 3).wait()
    escat_desc(3, 1, 3).start()
    gadd_desc(0, 1, 4).wait()
    escat_desc(0, 1, 4).start()
    escat_desc(1, 1, 1).wait()
    escat_desc(2, 1, 2).wait()
    escat_desc(3, 1, 3).wait()
    escat_desc(0, 1, 4).wait()
    plsc.subcore_barrier()

    pltpu.sync_copy(accum.at[pl.ds(s * ROWS_PER_TILE, ROWS_PER_TILE)],
                    out_hbm.at[c, pl.ds(s * ROWS_PER_TILE, ROWS_PER_TILE)])


_NB = 400
_GRID = N_NODES // _NB


def _mlp_body(u0_ref, u1_ref, x_ref, w1_ref, b1_ref, w2_ref, b2_ref,
              g_ref, bt_ref, o_ref):
    u = u0_ref[...] + u1_ref[...]
    h = jnp.dot(u, w1_ref[...], preferred_element_type=jnp.float32) + b1_ref[...]
    h = jnp.maximum(h, 0.0)
    h = jnp.dot(h, w2_ref[...], preferred_element_type=jnp.float32) + b2_ref[...]
    h = h + x_ref[...]
    mean = jnp.mean(h, axis=-1, keepdims=True)
    hc = h - mean
    var = jnp.mean(hc * hc, axis=-1, keepdims=True)
    o_ref[...] = hc * lax.rsqrt(var + 1e-5) * g_ref[...] + bt_ref[...]


_mlp = pl.pallas_call(
    _mlp_body,
    grid=(_GRID,),
    in_specs=[
        pl.BlockSpec((_NB, D), lambda i: (i, 0)),
        pl.BlockSpec((_NB, D), lambda i: (i, 0)),
        pl.BlockSpec((_NB, D), lambda i: (i, 0)),
        pl.BlockSpec((D, H), lambda i: (0, 0)),
        pl.BlockSpec((1, H), lambda i: (0, 0)),
        pl.BlockSpec((H, D), lambda i: (0, 0)),
        pl.BlockSpec((1, D), lambda i: (0, 0)),
        pl.BlockSpec((1, D), lambda i: (0, 0)),
        pl.BlockSpec((1, D), lambda i: (0, 0)),
    ],
    out_specs=pl.BlockSpec((_NB, D), lambda i: (i, 0)),
    out_shape=jax.ShapeDtypeStruct((N_NODES, D), jnp.float32),
)


def kernel(x, edge_index, edge_attr, W1, b1, W2, b2, gamma, beta):
    src = edge_index[0].astype(jnp.int32).reshape(NW, NBLK, K)
    dst = edge_index[1].astype(jnp.int32).reshape(NW, NBLK, K)
    pad = ((0, 0), (0, NBLK_PAD - NBLK), (0, 0))
    src = jnp.pad(src, pad)
    dst = jnp.pad(dst, pad)
    zeros = jnp.zeros((K, D), jnp.float32)
    u_parts = _sc_aggregate(x, src, dst, edge_attr, zeros)
    return _mlp(u_parts[0], u_parts[1], x,
                W1, b1.reshape(1, H), W2, b2.reshape(1, D),
                gamma.reshape(1, D), beta.reshape(1, D))
